# Initial kernel scaffold; baseline (speedup 1.0000x reference)
import jax
import jax.numpy as jnp
from jax.experimental import pallas as pl


def kernel(node_feats, edge_feats, edge_index, W_in, b_in, W_msg, b_msg, W_new, b_new, W_p1, b_p1, W_p2, b_p2):
    raise NotImplementedError("write your pallas kernel here")


# speedup vs baseline: 5.6817x; 5.6817x over previous
import functools

import jax
import jax.numpy as jnp
from jax import lax
from jax.experimental import pallas as pl
from jax.experimental.pallas import tpu as pltpu
from jax.experimental.pallas import tpu_sc as plsc

N = 10000
E = 320000
DN = 128
DE = 16
H = 64
C = 12
L = 3

NC = 2
NS = 16
NW = NC * NS
EPW = E // NW
K = 40
G = 5
S = G * K
NITER = EPW // S
RPT = 640
RPT_LAST = N - RPT * (NS - 1)
ZR = 80

ROW_BLK = 2000
NBLK = N // ROW_BLK
EBLK = 8000
NEBLK = E // EBLK


def _in_proj_body(nf, w_in, b_in, w_mh, h_out, hp_out):
    h = jnp.maximum(jnp.dot(nf[...], w_in[...],
                            preferred_element_type=jnp.float32) + b_in[...], 0.0)
    h_out[...] = h
    hp_out[...] = jnp.dot(h, w_mh[...], preferred_element_type=jnp.float32)


_in_proj = pl.pallas_call(
    _in_proj_body,
    grid=(NBLK,),
    in_specs=[
        pl.BlockSpec((ROW_BLK, DN), lambda i: (i, 0)),
        pl.BlockSpec((DN, H), lambda i: (0, 0)),
        pl.BlockSpec((1, H), lambda i: (0, 0)),
        pl.BlockSpec((H, H), lambda i: (0, 0)),
    ],
    out_specs=[
        pl.BlockSpec((ROW_BLK, H), lambda i: (i, 0)),
        pl.BlockSpec((ROW_BLK, H), lambda i: (i, 0)),
    ],
    out_shape=[
        jax.ShapeDtypeStruct((N, H), jnp.float32),
        jax.ShapeDtypeStruct((N, H), jnp.float32),
    ],
)


def _eproj_body(ef, w_me, b_msg, out):
    out[...] = jnp.dot(ef[...], w_me[...],
                       preferred_element_type=jnp.float32) + b_msg[...]


_eproj = pl.pallas_call(
    _eproj_body,
    grid=(NEBLK,),
    in_specs=[
        pl.BlockSpec((EBLK, DE), lambda i: (i, 0)),
        pl.BlockSpec((DE, H), lambda i: (0, 0)),
        pl.BlockSpec((1, H), lambda i: (0, 0)),
    ],
    out_specs=pl.BlockSpec((EBLK, H), lambda i: (i, 0)),
    out_shape=jax.ShapeDtypeStruct((E, H), jnp.float32),
)


def _update_body(h, p0, p1, w_h, w_m, b, w_mh, h_out, hp_out):
    m = p0[...] + p1[...]
    hn = jnp.maximum(
        jnp.dot(h[...], w_h[...], preferred_element_type=jnp.float32)
        + jnp.dot(m, w_m[...], preferred_element_type=jnp.float32) + b[...], 0.0)
    h_out[...] = hn
    hp_out[...] = jnp.dot(hn, w_mh[...], preferred_element_type=jnp.float32)


_update = pl.pallas_call(
    _update_body,
    grid=(NBLK,),
    in_specs=[
        pl.BlockSpec((ROW_BLK, H), lambda i: (i, 0)),
        pl.BlockSpec((ROW_BLK, H), lambda i: (i, 0)),
        pl.BlockSpec((ROW_BLK, H), lambda i: (i, 0)),
        pl.BlockSpec((H, H), lambda i: (0, 0)),
        pl.BlockSpec((H, H), lambda i: (0, 0)),
        pl.BlockSpec((1, H), lambda i: (0, 0)),
        pl.BlockSpec((H, H), lambda i: (0, 0)),
    ],
    out_specs=[
        pl.BlockSpec((ROW_BLK, H), lambda i: (i, 0)),
        pl.BlockSpec((ROW_BLK, H), lambda i: (i, 0)),
    ],
    out_shape=[
        jax.ShapeDtypeStruct((N, H), jnp.float32),
        jax.ShapeDtypeStruct((N, H), jnp.float32),
    ],
)


def _final_body(h, p0, p1, w_h, w_m, b, w_p1, b_p1, w_p2, b_p2, out, acc):
    i = pl.program_id(0)
    m = p0[...] + p1[...]
    hn = jnp.maximum(
        jnp.dot(h[...], w_h[...], preferred_element_type=jnp.float32)
        + jnp.dot(m, w_m[...], preferred_element_type=jnp.float32) + b[...], 0.0)
    part = jnp.sum(hn, axis=0, keepdims=True)

    @pl.when(i == 0)
    def _():
        acc[...] = part

    @pl.when(i > 0)
    def _():
        acc[...] = acc[...] + part

    @pl.when(i == NBLK - 1)
    def _():
        g = jnp.maximum(jnp.dot(acc[...], w_p1[...],
                                preferred_element_type=jnp.float32) + b_p1[...], 0.0)
        out[...] = jnp.dot(g, w_p2[...],
                           preferred_element_type=jnp.float32) + b_p2[...]


_final = pl.pallas_call(
    _final_body,
    grid=(NBLK,),
    in_specs=[
        pl.BlockSpec((ROW_BLK, H), lambda i: (i, 0)),
        pl.BlockSpec((ROW_BLK, H), lambda i: (i, 0)),
        pl.BlockSpec((ROW_BLK, H), lambda i: (i, 0)),
        pl.BlockSpec((H, H), lambda i: (0, 0)),
        pl.BlockSpec((H, H), lambda i: (0, 0)),
        pl.BlockSpec((1, H), lambda i: (0, 0)),
        pl.BlockSpec((H, H), lambda i: (0, 0)),
        pl.BlockSpec((1, H), lambda i: (0, 0)),
        pl.BlockSpec((H, C), lambda i: (0, 0)),
        pl.BlockSpec((1, C), lambda i: (0, 0)),
    ],
    out_specs=pl.BlockSpec((1, C), lambda i: (0, 0)),
    out_shape=jax.ShapeDtypeStruct((1, C), jnp.float32),
    scratch_shapes=[pltpu.VMEM((1, H), jnp.float32)],
)


_sc_mesh = plsc.VectorSubcoreMesh(core_axis_name="c", subcore_axis_name="s")


@functools.partial(
    pl.kernel,
    out_type=jax.ShapeDtypeStruct((NC, N, H), jnp.float32),
    mesh=_sc_mesh,
    scratch_types=[
        pltpu.VMEM_SHARED((N, H), jnp.float32),
        pltpu.VMEM_SHARED((N, H), jnp.float32),
        pltpu.VMEM((G, K), jnp.int32),
        pltpu.VMEM((G, K), jnp.int32),
        pltpu.VMEM((G, K), jnp.int32),
        pltpu.VMEM((G, K), jnp.int32),
        pltpu.VMEM((S, H), jnp.float32),
        pltpu.VMEM((S, H), jnp.float32),
        pltpu.VMEM((S, H), jnp.float32),
        pltpu.VMEM((ZR, H), jnp.float32),
        pltpu.SemaphoreType.DMA,
        pltpu.SemaphoreType.DMA,
        pltpu.SemaphoreType.DMA,
        pltpu.SemaphoreType.DMA,
        [pltpu.SemaphoreType.DMA] * G,
        [pltpu.SemaphoreType.DMA] * G,
        pltpu.SemaphoreType.DMA,
        pltpu.SemaphoreType.DMA,
    ],
    compiler_params=pltpu.CompilerParams(use_tc_tiling_on_sc=False),
)
def _msg_pass(hp_hbm, eproj_hbm, src_hbm, dst_hbm, out_hbm,
              hps, acc, srcb0, srcb1, dstb0, dstb1, eb0, eb1, rb, zb,
              si0, si1, se0, se1, sg0, sg1, sc0, sc1):
    cid = lax.axis_index("c")
    sid = lax.axis_index("s")
    row0 = sid * RPT
    base = (cid * NS + sid) * EPW

    slots = ((srcb0, dstb0, eb0, rb, si0, se0, sg0, sc0),
             (srcb1, dstb1, eb1, rb, si1, se1, sg1, sc1))

    @pl.when(sid < NS - 1)
    def _():
        pltpu.sync_copy(hp_hbm.at[pl.ds(row0, RPT)], hps.at[pl.ds(row0, RPT)])

    @pl.when(sid == NS - 1)
    def _():
        pltpu.sync_copy(hp_hbm.at[pl.ds((NS - 1) * RPT, RPT_LAST)],
                        hps.at[pl.ds((NS - 1) * RPT, RPT_LAST)])

    zero = jnp.zeros((16,), jnp.float32)

    def zrow(k, carry):
        for c in range(H // 16):
            zb[k, pl.ds(c * 16, 16)] = zero
        return carry

    lax.fori_loop(0, ZR, zrow, 0)

    @pl.when(sid < NS - 1)
    def _():
        for r in range(RPT // ZR):
            pltpu.sync_copy(zb, acc.at[pl.ds(row0 + r * ZR, ZR)])

    @pl.when(sid == NS - 1)
    def _():
        for r in range(RPT_LAST // ZR):
            pltpu.sync_copy(zb, acc.at[pl.ds((NS - 1) * RPT + r * ZR, ZR)])

    plsc.subcore_barrier()

    def fetch(i, p):
        srcb, dstb, eb, rb, si, se, sg, sc = slots[p]
        off = base + i * S
        for g in range(G):
            pltpu.async_copy(src_hbm.at[pl.ds(off + g * K, K)], srcb.at[g], si)
            pltpu.async_copy(dst_hbm.at[pl.ds(off + g * K, K)], dstb.at[g], si)
        pltpu.async_copy(eproj_hbm.at[pl.ds(off, S)], eb, se)

    def drain_scatter(i, p):
        srcb, dstb, eb, rb, si, se, sg, sc = slots[p]
        for g in range(G):
            pltpu.make_async_copy(eb.at[pl.ds(g * K, K)],
                                  acc.at[dstb.at[g]], sc).wait()

    def process(i, p):
        srcb, dstb, eb, rb, si, se, sg, sc = slots[p]
        q = 1 - p
        off = base + i * S

        @pl.when(i >= 1)
        def _():
            drain_scatter(i - 1, q)

        @pl.when(i + 1 < NITER)
        def _():
            fetch(i + 1, q)

        for g in range(G):
            pltpu.make_async_copy(src_hbm.at[pl.ds(off + g * K, K)],
                                  srcb.at[g], si).wait()
            pltpu.make_async_copy(dst_hbm.at[pl.ds(off + g * K, K)],
                                  dstb.at[g], si).wait()
        for g in range(G):
            pltpu.async_copy(hps.at[srcb.at[g]],
                             rb.at[pl.ds(g * K, K)], sg[g])

        pltpu.make_async_copy(eproj_hbm.at[pl.ds(off, S)], eb, se).wait()

        for g in range(G):
            pltpu.make_async_copy(hps.at[srcb.at[g]],
                                  rb.at[pl.ds(g * K, K)], sg[g]).wait()
            r0 = g * K

            def crow(k, c2):
                for c in range(H // 16):
                    sl = pl.ds(c * 16, 16)
                    eb[r0 + k, sl] = jnp.maximum(rb[r0 + k, sl] + eb[r0 + k, sl],
                                                 0.0)
                return c2

            lax.fori_loop(0, K, crow, 0)
            pltpu.async_copy(eb.at[pl.ds(g * K, K)],
                             acc.at[dstb.at[g]], sc, add=True)

    fetch(0, 0)

    def chunk(i, carry):
        @pl.when(i % 2 == 0)
        def _():
            process(i, 0)

        @pl.when(i % 2 == 1)
        def _():
            process(i, 1)

        return carry

    lax.fori_loop(0, NITER, chunk, 0)
    drain_scatter(NITER - 1, (NITER - 1) % 2)

    plsc.subcore_barrier()

    @pl.when(sid < NS - 1)
    def _():
        pltpu.sync_copy(acc.at[pl.ds(row0, RPT)],
                        out_hbm.at[cid, pl.ds(row0, RPT)])

    @pl.when(sid == NS - 1)
    def _():
        pltpu.sync_copy(acc.at[pl.ds((NS - 1) * RPT, RPT_LAST)],
                        out_hbm.at[cid, pl.ds((NS - 1) * RPT, RPT_LAST)])


def kernel(node_feats, edge_feats, edge_index, W_in, b_in, W_msg, b_msg,
           W_new, b_new, W_p1, b_p1, W_p2, b_p2):
    src = edge_index[0]
    dst = edge_index[1]
    W_mh = W_msg[:H]
    W_me = W_msg[H:]
    W_nh = W_new[:H]
    W_nm = W_new[H:]
    b_in2 = b_in.reshape(1, H)
    b_msg2 = b_msg.reshape(1, H)
    b_new2 = b_new.reshape(1, H)
    b_p12 = b_p1.reshape(1, H)
    b_p22 = b_p2.reshape(1, C)

    h, hp = _in_proj(node_feats, W_in, b_in2, W_mh)
    ep = _eproj(edge_feats, W_me, b_msg2)

    for layer in range(L):
        parts = _msg_pass(hp, ep, src, dst)
        p0 = parts[0]
        p1 = parts[1]
        if layer < L - 1:
            h, hp = _update(h, p0, p1, W_nh, W_nm, b_new2, W_mh)
        else:
            out = _final(h, p0, p1, W_nh, W_nm, b_new2,
                         W_p1, b_p12, W_p2, b_p22)
    return out
